# Initial kernel scaffold; baseline (speedup 1.0000x reference)
import jax
import jax.numpy as jnp
from jax.experimental import pallas as pl


def kernel(x, weights, fc1_weight, fc2_weight, indices, counts):
    raise NotImplementedError("write your pallas kernel here")


# speedup vs baseline: 2.4972x; 2.4972x over previous
import functools

import jax
import jax.numpy as jnp
from jax import lax
from jax.experimental import pallas as pl

D_MODEL = 1024
D_INTER = 512
N_EXPERTS = 8
TOP_K = 2
N_TOKENS = 2048

BT = 512


def _dense_body(x_ref, w_ref, idx_ref, w1_ref, w2_ref, out_ref):
    e = pl.program_id(1)
    x = x_ref[...]
    w1 = w1_ref[0]
    w2 = w2_ref[0]
    h = lax.dot_general(x, w1, (((1,), (1,)), ((), ())),
                        preferred_element_type=jnp.float32)
    gate = h[:, :D_INTER]
    up = h[:, D_INTER:]
    a = gate * jax.nn.sigmoid(gate) * up
    y = lax.dot_general(a, w2, (((1,), (1,)), ((), ())),
                        preferred_element_type=jnp.float32)
    idx = idx_ref[...]
    w = w_ref[...]
    wm = jnp.sum(jnp.where(idx == e, w, 0.0), axis=1)
    contrib = wm[:, None] * y

    @pl.when(e == 0)
    def _():
        out_ref[...] = contrib

    @pl.when(e > 0)
    def _():
        out_ref[...] += contrib


def kernel(x, weights, fc1_weight, fc2_weight, indices, counts):
    del counts
    n_tb = N_TOKENS // BT
    return pl.pallas_call(
        _dense_body,
        grid=(n_tb, N_EXPERTS),
        in_specs=[
            pl.BlockSpec((BT, D_MODEL), lambda i, e: (i, 0)),
            pl.BlockSpec((BT, TOP_K), lambda i, e: (i, 0)),
            pl.BlockSpec((BT, TOP_K), lambda i, e: (i, 0)),
            pl.BlockSpec((1, 2 * D_INTER, D_MODEL), lambda i, e: (e, 0, 0)),
            pl.BlockSpec((1, D_MODEL, D_INTER), lambda i, e: (e, 0, 0)),
        ],
        out_specs=pl.BlockSpec((BT, D_MODEL), lambda i, e: (i, 0)),
        out_shape=jax.ShapeDtypeStruct((N_TOKENS, D_MODEL), jnp.float32),
    )(x, weights, indices, fc1_weight, fc2_weight)
